# baseline (device time: 40186 ns/iter reference)
import jax
import jax.numpy as jnp
from jax import lax
from jax.experimental import pallas as pl
from jax.experimental.pallas import tpu as pltpu

N_DEV = 16
N_LAYERS = 3


def kernel(x, Win0, Wout0, Win1, Wout1, Win2, Wout2):
    b, d = x.shape
    hdim = Win0.shape[1]
    rows = b // N_DEV

    def body(
        x_hbm,
        win0_hbm,
        wout0_hbm,
        win1_hbm,
        wout1_hbm,
        win2_hbm,
        wout2_hbm,
        out_ref,
        xv_ref,
        wins_ref,
        wouts_ref,
        mine_ref,
        red_ref,
        rs_ref,
        ag_ref,
        load_sems,
        rs_send_sems,
        rs_recv_sems,
        ag_send_sems,
        ag_recv_sems,
    ):
        my = lax.axis_index("i")
        win_hbms = [win0_hbm, win1_hbm, win2_hbm]
        wout_hbms = [wout0_hbm, wout1_hbm, wout2_hbm]

        loads = []
        cp = pltpu.make_async_copy(x_hbm, xv_ref, load_sems.at[0])
        cp.start()
        loads.append(cp)
        for l in range(N_LAYERS):
            cp = pltpu.make_async_copy(
                win_hbms[l], wins_ref.at[l], load_sems.at[1 + 2 * l]
            )
            cp.start()
            loads.append(cp)
            cp = pltpu.make_async_copy(
                wout_hbms[l], wouts_ref.at[l], load_sems.at[2 + 2 * l]
            )
            cp.start()
            loads.append(cp)

        loads[0].wait()
        acc = xv_ref[...]
        for layer in range(N_LAYERS):
            loads[1 + 2 * layer].wait()
            h = jnp.maximum(
                jnp.dot(
                    acc, wins_ref[layer], preferred_element_type=jnp.float32
                ),
                0.0,
            )
            loads[2 + 2 * layer].wait()
            partial = jnp.dot(
                h, wouts_ref[layer], preferred_element_type=jnp.float32
            )
            mine_ref[...] = partial

            if layer == 0:
                barrier_sem = pltpu.get_barrier_semaphore()
                for o in range(1, N_DEV):
                    pl.semaphore_signal(
                        barrier_sem,
                        inc=1,
                        device_id=(lax.rem(my + o, N_DEV),),
                        device_id_type=pl.DeviceIdType.MESH,
                    )
                pl.semaphore_wait(barrier_sem, N_DEV - 1)

            rs_sends = []
            for o in range(1, N_DEV):
                tgt = lax.rem(my + o, N_DEV)
                rdma = pltpu.make_async_remote_copy(
                    src_ref=mine_ref.at[pl.ds(tgt * rows, rows), :],
                    dst_ref=rs_ref.at[layer, o],
                    send_sem=rs_send_sems.at[layer, o],
                    recv_sem=rs_recv_sems.at[layer, o],
                    device_id=(tgt,),
                    device_id_type=pl.DeviceIdType.MESH,
                )
                rdma.start()
                rs_sends.append(rdma)

            red = mine_ref[pl.ds(my * rows, rows), :]
            for o in range(1, N_DEV):
                recv = pltpu.make_async_remote_copy(
                    src_ref=rs_ref.at[layer, o],
                    dst_ref=rs_ref.at[layer, o],
                    send_sem=rs_send_sems.at[layer, o],
                    recv_sem=rs_recv_sems.at[layer, o],
                    device_id=(my,),
                    device_id_type=pl.DeviceIdType.MESH,
                )
                recv.wait_recv()
                red = red + rs_ref[layer, o]

            red_ref[...] = red
            ag_ref[layer, pl.ds(my * rows, rows), :] = red
            ag_sends = []
            for o in range(1, N_DEV):
                tgt = lax.rem(my + o, N_DEV)
                rdma = pltpu.make_async_remote_copy(
                    src_ref=red_ref,
                    dst_ref=ag_ref.at[layer, pl.ds(my * rows, rows), :],
                    send_sem=ag_send_sems.at[layer, o],
                    recv_sem=ag_recv_sems.at[layer, o],
                    device_id=(tgt,),
                    device_id_type=pl.DeviceIdType.MESH,
                )
                rdma.start()
                ag_sends.append(rdma)

            for o in range(1, N_DEV):
                recv = pltpu.make_async_remote_copy(
                    src_ref=red_ref,
                    dst_ref=ag_ref.at[layer, pl.ds(my * rows, rows), :],
                    send_sem=ag_send_sems.at[layer, o],
                    recv_sem=ag_recv_sems.at[layer, o],
                    device_id=(my,),
                    device_id_type=pl.DeviceIdType.MESH,
                )
                recv.wait_recv()

            acc = ag_ref[layer]

            for rdma in rs_sends:
                rdma.wait_send()
            for rdma in ag_sends:
                rdma.wait_send()

        out_ref[...] = acc

    return pl.pallas_call(
        body,
        out_shape=jax.ShapeDtypeStruct((b, d), jnp.float32),
        in_specs=[pl.BlockSpec(memory_space=pl.ANY)] * 7,
        out_specs=pl.BlockSpec(memory_space=pltpu.VMEM),
        scratch_shapes=[
            pltpu.VMEM((b, d), jnp.float32),
            pltpu.VMEM((N_LAYERS, d, hdim), jnp.float32),
            pltpu.VMEM((N_LAYERS, hdim, d), jnp.float32),
            pltpu.VMEM((b, d), jnp.float32),
            pltpu.VMEM((rows, d), jnp.float32),
            pltpu.VMEM((N_LAYERS, N_DEV, rows, d), jnp.float32),
            pltpu.VMEM((N_LAYERS, b, d), jnp.float32),
            pltpu.SemaphoreType.DMA((1 + 2 * N_LAYERS,)),
            pltpu.SemaphoreType.DMA((N_LAYERS, N_DEV)),
            pltpu.SemaphoreType.DMA((N_LAYERS, N_DEV)),
            pltpu.SemaphoreType.DMA((N_LAYERS, N_DEV)),
            pltpu.SemaphoreType.DMA((N_LAYERS, N_DEV)),
        ],
        compiler_params=pltpu.CompilerParams(collective_id=0),
    )(x, Win0, Wout0, Win1, Wout1, Win2, Wout2)


# device time: 33657 ns/iter; 1.1940x vs baseline; 1.1940x over previous
import jax
import jax.numpy as jnp
from jax import lax
from jax.experimental import pallas as pl
from jax.experimental.pallas import tpu as pltpu

N_DEV = 16
N_LAYERS = 3


def kernel(x, Win0, Wout0, Win1, Wout1, Win2, Wout2):
    b, d = x.shape
    hdim = Win0.shape[1]
    rows = b // N_DEV

    def body(
        x_hbm,
        win0_hbm,
        wout0_hbm,
        win1_hbm,
        wout1_hbm,
        win2_hbm,
        wout2_hbm,
        out_ref,
        xv_ref,
        wins_ref,
        wouts_ref,
        mine_ref,
        red_ref,
        rs_ref,
        ag_ref,
        load_sems,
        rs_send_sems,
        rs_recv_sems,
        ag_send_sems,
        ag_recv_sems,
    ):
        my = lax.axis_index("i")
        win_hbms = [win0_hbm, win1_hbm, win2_hbm]
        wout_hbms = [wout0_hbm, wout1_hbm, wout2_hbm]

        loads = []
        cp = pltpu.make_async_copy(x_hbm, xv_ref, load_sems.at[0])
        cp.start()
        loads.append(cp)
        for l in range(N_LAYERS):
            cp = pltpu.make_async_copy(
                win_hbms[l], wins_ref.at[l], load_sems.at[1 + 2 * l]
            )
            cp.start()
            loads.append(cp)
            cp = pltpu.make_async_copy(
                wout_hbms[l], wouts_ref.at[l], load_sems.at[2 + 2 * l]
            )
            cp.start()
            loads.append(cp)

        loads[0].wait()
        acc = xv_ref[...]
        for layer in range(N_LAYERS):
            loads[1 + 2 * layer].wait()
            h = jnp.maximum(
                jnp.dot(
                    acc, wins_ref[layer], preferred_element_type=jnp.float32
                ),
                0.0,
            )
            loads[2 + 2 * layer].wait()
            partial = jnp.dot(
                h, wouts_ref[layer], preferred_element_type=jnp.float32
            )
            mine_ref[...] = partial

            if layer == 0:
                barrier_sem = pltpu.get_barrier_semaphore()
                for o in range(1, N_DEV):
                    pl.semaphore_signal(
                        barrier_sem,
                        inc=1,
                        device_id=(lax.rem(my + o, N_DEV),),
                        device_id_type=pl.DeviceIdType.MESH,
                    )
                pl.semaphore_wait(barrier_sem, N_DEV - 1)

            rs_sends = []
            for o in range(1, N_DEV):
                tgt = lax.rem(my + o, N_DEV)
                rdma = pltpu.make_async_remote_copy(
                    src_ref=mine_ref.at[pl.ds(tgt * rows, rows), :],
                    dst_ref=rs_ref.at[layer, o],
                    send_sem=rs_send_sems.at[layer, o],
                    recv_sem=rs_recv_sems.at[layer, o],
                    device_id=(tgt,),
                    device_id_type=pl.DeviceIdType.MESH,
                )
                rdma.start()
                rs_sends.append(rdma)

            red = mine_ref[pl.ds(my * rows, rows), :]
            for o in range(1, N_DEV):
                recv = pltpu.make_async_remote_copy(
                    src_ref=rs_ref.at[layer, o],
                    dst_ref=rs_ref.at[layer, o],
                    send_sem=rs_send_sems.at[layer, o],
                    recv_sem=rs_recv_sems.at[layer, o],
                    device_id=(my,),
                    device_id_type=pl.DeviceIdType.MESH,
                )
                recv.wait_recv()
                red = red + rs_ref[layer, o]

            red_ref[...] = red
            ag_ref[layer, pl.ds(my * rows, rows), :] = red
            ag_sends = []
            for o in range(1, N_DEV):
                tgt = lax.rem(my + o, N_DEV)
                rdma = pltpu.make_async_remote_copy(
                    src_ref=red_ref,
                    dst_ref=ag_ref.at[layer, pl.ds(my * rows, rows), :],
                    send_sem=ag_send_sems.at[layer, o],
                    recv_sem=ag_recv_sems.at[layer, o],
                    device_id=(tgt,),
                    device_id_type=pl.DeviceIdType.MESH,
                )
                rdma.start()
                ag_sends.append(rdma)

            for o in range(1, N_DEV):
                recv = pltpu.make_async_remote_copy(
                    src_ref=red_ref,
                    dst_ref=ag_ref.at[layer, pl.ds(my * rows, rows), :],
                    send_sem=ag_send_sems.at[layer, o],
                    recv_sem=ag_recv_sems.at[layer, o],
                    device_id=(my,),
                    device_id_type=pl.DeviceIdType.MESH,
                )
                recv.wait_recv()

            acc = ag_ref[layer]

            for rdma in rs_sends:
                rdma.wait_send()
            for rdma in ag_sends:
                rdma.wait_send()

        out_ref[...] = acc

    return pl.pallas_call(
        body,
        out_shape=jax.ShapeDtypeStruct((b, d), jnp.float32),
        in_specs=[pl.BlockSpec(memory_space=pltpu.MemorySpace.HBM)] * 7,
        out_specs=pl.BlockSpec(memory_space=pltpu.VMEM),
        scratch_shapes=[
            pltpu.VMEM((b, d), jnp.float32),
            pltpu.VMEM((N_LAYERS, d, hdim), jnp.float32),
            pltpu.VMEM((N_LAYERS, hdim, d), jnp.float32),
            pltpu.VMEM((b, d), jnp.float32),
            pltpu.VMEM((rows, d), jnp.float32),
            pltpu.VMEM((N_LAYERS, N_DEV, rows, d), jnp.float32),
            pltpu.VMEM((N_LAYERS, b, d), jnp.float32),
            pltpu.SemaphoreType.DMA((1 + 2 * N_LAYERS,)),
            pltpu.SemaphoreType.DMA((N_LAYERS, N_DEV)),
            pltpu.SemaphoreType.DMA((N_LAYERS, N_DEV)),
            pltpu.SemaphoreType.DMA((N_LAYERS, N_DEV)),
            pltpu.SemaphoreType.DMA((N_LAYERS, N_DEV)),
        ],
        compiler_params=pltpu.CompilerParams(collective_id=0),
    )(
        *(
            pltpu.with_memory_space_constraint(a, pltpu.MemorySpace.HBM)
            for a in (x, Win0, Wout0, Win1, Wout1, Win2, Wout2)
        )
    )


# device time: 29572 ns/iter; 1.3589x vs baseline; 1.1381x over previous
import jax
import jax.numpy as jnp
from jax import lax
from jax.experimental import pallas as pl
from jax.experimental.pallas import tpu as pltpu

N_DEV = 16
N_LAYERS = 3


def kernel(x, Win0, Wout0, Win1, Wout1, Win2, Wout2):
    b, d = x.shape
    hdim = Win0.shape[1]
    rows = b // N_DEV

    def body(
        x_hbm,
        win0_hbm,
        wout0_hbm,
        win1_hbm,
        wout1_hbm,
        win2_hbm,
        wout2_hbm,
        out_ref,
        xv_ref,
        wins_ref,
        wouts_ref,
        mine_ref,
        red_ref,
        rs_ref,
        ag_ref,
        load_sems,
        rs_send_sems,
        rs_recv_sems,
        ag_send_sems,
        ag_recv_sems,
    ):
        my = lax.axis_index("i")
        win_hbms = [win0_hbm, win1_hbm, win2_hbm]
        wout_hbms = [wout0_hbm, wout1_hbm, wout2_hbm]

        barrier_sem = pltpu.get_barrier_semaphore()
        for o in range(1, N_DEV):
            pl.semaphore_signal(
                barrier_sem,
                inc=1,
                device_id=(lax.rem(my + o, N_DEV),),
                device_id_type=pl.DeviceIdType.MESH,
            )

        loads = []
        cp = pltpu.make_async_copy(x_hbm, xv_ref, load_sems.at[0])
        cp.start()
        loads.append(cp)
        for l in range(N_LAYERS):
            cp = pltpu.make_async_copy(
                win_hbms[l], wins_ref.at[l], load_sems.at[1 + 2 * l]
            )
            cp.start()
            loads.append(cp)
            cp = pltpu.make_async_copy(
                wout_hbms[l], wouts_ref.at[l], load_sems.at[2 + 2 * l]
            )
            cp.start()
            loads.append(cp)

        loads[0].wait()
        acc = xv_ref[...]
        for layer in range(N_LAYERS):
            loads[1 + 2 * layer].wait()
            h = jnp.maximum(
                jnp.dot(
                    acc, wins_ref[layer], preferred_element_type=jnp.float32
                ),
                0.0,
            )
            loads[2 + 2 * layer].wait()
            partial = jnp.dot(
                h, wouts_ref[layer], preferred_element_type=jnp.float32
            )
            mine_ref[...] = partial

            if layer == 0:
                pl.semaphore_wait(barrier_sem, N_DEV - 1)

            rs_sends = []
            for o in range(1, N_DEV):
                tgt = lax.rem(my + o, N_DEV)
                rdma = pltpu.make_async_remote_copy(
                    src_ref=mine_ref.at[pl.ds(tgt * rows, rows), :],
                    dst_ref=rs_ref.at[layer, o],
                    send_sem=rs_send_sems.at[layer, o],
                    recv_sem=rs_recv_sems.at[layer, o],
                    device_id=(tgt,),
                    device_id_type=pl.DeviceIdType.MESH,
                )
                rdma.start()
                rs_sends.append(rdma)

            red = mine_ref[pl.ds(my * rows, rows), :]
            for o in range(1, N_DEV):
                recv = pltpu.make_async_remote_copy(
                    src_ref=rs_ref.at[layer, o],
                    dst_ref=rs_ref.at[layer, o],
                    send_sem=rs_send_sems.at[layer, o],
                    recv_sem=rs_recv_sems.at[layer, o],
                    device_id=(my,),
                    device_id_type=pl.DeviceIdType.MESH,
                )
                recv.wait_recv()
                red = red + rs_ref[layer, o]

            red_ref[...] = red
            ag_ref[layer, pl.ds(my * rows, rows), :] = red
            ag_sends = []
            for o in range(1, N_DEV):
                tgt = lax.rem(my + o, N_DEV)
                rdma = pltpu.make_async_remote_copy(
                    src_ref=red_ref,
                    dst_ref=ag_ref.at[layer, pl.ds(my * rows, rows), :],
                    send_sem=ag_send_sems.at[layer, o],
                    recv_sem=ag_recv_sems.at[layer, o],
                    device_id=(tgt,),
                    device_id_type=pl.DeviceIdType.MESH,
                )
                rdma.start()
                ag_sends.append(rdma)

            for o in range(1, N_DEV):
                recv = pltpu.make_async_remote_copy(
                    src_ref=red_ref,
                    dst_ref=ag_ref.at[layer, pl.ds(my * rows, rows), :],
                    send_sem=ag_send_sems.at[layer, o],
                    recv_sem=ag_recv_sems.at[layer, o],
                    device_id=(my,),
                    device_id_type=pl.DeviceIdType.MESH,
                )
                recv.wait_recv()

            acc = ag_ref[layer]

            for rdma in rs_sends:
                rdma.wait_send()
            for rdma in ag_sends:
                rdma.wait_send()

        out_ref[...] = acc

    return pl.pallas_call(
        body,
        out_shape=jax.ShapeDtypeStruct((b, d), jnp.float32),
        in_specs=[pl.BlockSpec(memory_space=pltpu.MemorySpace.HBM)] * 7,
        out_specs=pl.BlockSpec(memory_space=pltpu.VMEM),
        scratch_shapes=[
            pltpu.VMEM((b, d), jnp.float32),
            pltpu.VMEM((N_LAYERS, d, hdim), jnp.float32),
            pltpu.VMEM((N_LAYERS, hdim, d), jnp.float32),
            pltpu.VMEM((b, d), jnp.float32),
            pltpu.VMEM((rows, d), jnp.float32),
            pltpu.VMEM((N_LAYERS, N_DEV, rows, d), jnp.float32),
            pltpu.VMEM((N_LAYERS, b, d), jnp.float32),
            pltpu.SemaphoreType.DMA((1 + 2 * N_LAYERS,)),
            pltpu.SemaphoreType.DMA((N_LAYERS, N_DEV)),
            pltpu.SemaphoreType.DMA((N_LAYERS, N_DEV)),
            pltpu.SemaphoreType.DMA((N_LAYERS, N_DEV)),
            pltpu.SemaphoreType.DMA((N_LAYERS, N_DEV)),
        ],
        compiler_params=pltpu.CompilerParams(collective_id=0),
    )(
        *(
            pltpu.with_memory_space_constraint(a, pltpu.MemorySpace.HBM)
            for a in (x, Win0, Wout0, Win1, Wout1, Win2, Wout2)
        )
    )
